# Initial kernel scaffold; baseline (speedup 1.0000x reference)
import jax
import jax.numpy as jnp
from jax.experimental import pallas as pl


def kernel(ids, weight):
    raise NotImplementedError("write your pallas kernel here")


# speedup vs baseline: 2.6703x; 2.6703x over previous
import jax
import jax.numpy as jnp
from jax import lax
from jax.experimental import pallas as pl
from jax.experimental.pallas import tpu as pltpu
from jax.experimental.pallas import tpu_sc as plsc

_W = 128
_NW = 32
_LANES = 16


def kernel(ids, weight):
    n_ids = ids.shape[0] * ids.shape[1]
    vocab, dim = weight.shape
    pack = 128 // dim
    flat_ids = ids.reshape(n_ids)
    w_packed = weight.reshape(vocab // pack, 128)

    n_per_w = n_ids // _NW
    chunks = n_per_w // _W

    mesh = plsc.VectorSubcoreMesh(core_axis_name="core", subcore_axis_name="subcore")

    @pl.kernel(
        out_type=jax.ShapeDtypeStruct((n_ids * dim,), weight.dtype),
        mesh=mesh,
        scratch_types=[
            pltpu.VMEM((_W,), jnp.int32),
            pltpu.VMEM((_W,), jnp.int32),
            pltpu.VMEM((_W, 128), jnp.float32),
            pltpu.VMEM((_W * dim,), jnp.float32),
            pltpu.SemaphoreType.DMA,
        ],
    )
    def gather_kernel(w_hbm, i_hbm, o_hbm, idx_v, pidx_v, rows_v, out_v, sem):
        wid = lax.axis_index("subcore") * 2 + lax.axis_index("core")
        base_w = wid * n_per_w

        @pl.loop(0, chunks)
        def _(g):
            base = base_w + g * _W
            pltpu.sync_copy(i_hbm.at[pl.ds(base, _W)], idx_v)

            @pl.loop(0, _W, step=_LANES)
            def _(j):
                pidx_v[pl.ds(j, _LANES)] = idx_v[pl.ds(j, _LANES)] >> 2

            pltpu.async_copy(w_hbm.at[pidx_v], rows_v, sem).wait()

            @pl.loop(0, _W, step=_LANES)
            def _(b):
                offs = (idx_v[pl.ds(b, _LANES)] & (pack - 1)) * dim
                for j in range(_LANES):
                    r = b + j
                    off = offs[j]
                    out_v[pl.ds(r * dim, _LANES)] = rows_v[r, pl.ds(off, _LANES)]
                    out_v[pl.ds(r * dim + _LANES, _LANES)] = rows_v[
                        r, pl.ds(off + _LANES, _LANES)
                    ]

            pltpu.sync_copy(out_v, o_hbm.at[pl.ds(base * dim, _W * dim)])

    out = gather_kernel(w_packed, flat_ids)
    return out.reshape(ids.shape[0], ids.shape[1], dim)


# speedup vs baseline: 3.7353x; 1.3988x over previous
import jax
import jax.numpy as jnp
from jax import lax
from jax.experimental import pallas as pl
from jax.experimental.pallas import tpu as pltpu
from jax.experimental.pallas import tpu_sc as plsc

_W = 128
_SUPER = 16
_NW = 32
_LANES = 16


def kernel(ids, weight):
    n_ids = ids.shape[0] * ids.shape[1]
    vocab, dim = weight.shape
    pack = 128 // dim
    flat_ids = ids.reshape(n_ids)
    w_packed = weight.reshape(vocab // pack, 128)

    n_per_w = n_ids // _NW
    n_super = _SUPER * _W
    supers = n_per_w // n_super

    mesh = plsc.VectorSubcoreMesh(core_axis_name="core", subcore_axis_name="subcore")

    @pl.kernel(
        out_type=jax.ShapeDtypeStruct((n_ids * dim,), weight.dtype),
        mesh=mesh,
        scratch_types=[
            pltpu.VMEM((n_super,), jnp.int32),
            pltpu.VMEM((n_super,), jnp.int32),
            pltpu.VMEM((_W, 128), jnp.float32),
            pltpu.VMEM((_W, 128), jnp.float32),
            pltpu.VMEM((_W * dim,), jnp.float32),
            pltpu.VMEM((_W * dim,), jnp.float32),
            pltpu.SemaphoreType.DMA,
            pltpu.SemaphoreType.DMA,
            pltpu.SemaphoreType.DMA,
            pltpu.SemaphoreType.DMA,
        ],
    )
    def gather_kernel(
        w_hbm, i_hbm, o_hbm, idx_v, pidx_v, rows0, rows1, out0, out1, g0, g1, o0, o1
    ):
        wid = lax.axis_index("subcore") * 2 + lax.axis_index("core")
        base_w = wid * n_per_w
        rows = (rows0, rows1)
        outs = (out0, out1)
        gsems = (g0, g1)
        osems = (o0, o1)

        def start_gather(c, slot):
            pltpu.async_copy(
                w_hbm.at[pidx_v.at[pl.ds(c * _W, _W)]], rows[slot], gsems[slot]
            )

        def extract(c, slot):
            @pl.loop(0, _W, step=_LANES)
            def _(b):
                offs = (idx_v[pl.ds(c * _W + b, _LANES)] & (pack - 1)) * dim
                for j in range(_LANES):
                    r = b + j
                    off = offs[j]
                    outs[slot][pl.ds(r * dim, _LANES)] = rows[slot][
                        r, pl.ds(off, _LANES)
                    ]
                    outs[slot][pl.ds(r * dim + _LANES, _LANES)] = rows[slot][
                        r, pl.ds(off + _LANES, _LANES)
                    ]

        @pl.loop(0, supers)
        def _(s):
            base = base_w + s * n_super
            pltpu.sync_copy(i_hbm.at[pl.ds(base, n_super)], idx_v)

            @pl.loop(0, n_super, step=_LANES)
            def _(j):
                pidx_v[pl.ds(j, _LANES)] = idx_v[pl.ds(j, _LANES)] >> 2

            start_gather(0, 0)

            @pl.loop(0, _SUPER, step=2)
            def _(c):
                for u in range(2):
                    cc = c + u
                    slot = u
                    other = 1 - u

                    @pl.when(cc + 1 < _SUPER)
                    def _():
                        start_gather(cc + 1, other)

                    @pl.when((s > 0) | (cc >= 2))
                    def _():
                        pltpu.make_async_copy(
                            outs[slot], o_hbm.at[pl.ds(0, _W * dim)], osems[slot]
                        ).wait()

                    pltpu.make_async_copy(
                        w_hbm.at[pidx_v.at[pl.ds(cc * _W, _W)]],
                        rows[slot],
                        gsems[slot],
                    ).wait()
                    extract(cc, slot)
                    pltpu.async_copy(
                        outs[slot],
                        o_hbm.at[pl.ds((base + cc * _W) * dim, _W * dim)],
                        osems[slot],
                    )

        for slot in range(2):
            pltpu.make_async_copy(
                outs[slot], o_hbm.at[pl.ds(0, _W * dim)], osems[slot]
            ).wait()

    out = gather_kernel(w_packed, flat_ids)
    return out.reshape(ids.shape[0], ids.shape[1], dim)


# speedup vs baseline: 4.1367x; 1.1075x over previous
import jax
import jax.numpy as jnp
from jax import lax
from jax.experimental import pallas as pl
from jax.experimental.pallas import tpu as pltpu
from jax.experimental.pallas import tpu_sc as plsc

_W = 128
_SUPER = 16
_NW = 32
_LANES = 16
_NBUF = 4


def kernel(ids, weight):
    batch, seq = ids.shape
    n_ids = batch * seq
    vocab, dim = weight.shape
    pack = 128 // dim
    flat_ids = ids.T.reshape(n_ids)
    w_packed = weight.reshape(vocab // pack, 128)

    n_per_w = n_ids // _NW
    n_super = _SUPER * _W
    supers = n_per_w // n_super

    mesh = plsc.VectorSubcoreMesh(core_axis_name="core", subcore_axis_name="subcore")

    @pl.kernel(
        out_type=jax.ShapeDtypeStruct((n_ids * dim,), weight.dtype),
        mesh=mesh,
        scratch_types=[
            pltpu.VMEM((n_super,), jnp.int32),
            pltpu.VMEM((n_super,), jnp.int32),
        ]
        + [pltpu.VMEM((_W, 128), jnp.float32)] * _NBUF
        + [pltpu.VMEM((_W * dim,), jnp.float32)] * 2
        + [pltpu.SemaphoreType.DMA] * (_NBUF + 2),
    )
    def gather_kernel(w_hbm, i_hbm, o_hbm, idx_v, pidx_v, *bufs):
        rows = bufs[:_NBUF]
        outs = bufs[_NBUF : _NBUF + 2]
        gsems = bufs[_NBUF + 2 : 2 * _NBUF + 2]
        osems = bufs[2 * _NBUF + 2 :]
        wid = lax.axis_index("subcore") * 2 + lax.axis_index("core")
        base_w = wid * n_per_w

        def start_gather(c, slot):
            pltpu.async_copy(
                w_hbm.at[pidx_v.at[pl.ds(c * _W, _W)]], rows[slot], gsems[slot]
            )

        def wait_gather(c, slot):
            pltpu.make_async_copy(
                w_hbm.at[pidx_v.at[pl.ds(c * _W, _W)]], rows[slot], gsems[slot]
            ).wait()

        def extract(c, slot, oslot):
            @pl.loop(0, _W, step=_LANES)
            def _(b):
                offs = (idx_v[pl.ds(c * _W + b, _LANES)] & (pack - 1)) * dim
                for j in range(_LANES):
                    r = b + j
                    off = offs[j]
                    outs[oslot][pl.ds(r * dim, _LANES)] = rows[slot][
                        r, pl.ds(off, _LANES)
                    ]
                    outs[oslot][pl.ds(r * dim + _LANES, _LANES)] = rows[slot][
                        r, pl.ds(off + _LANES, _LANES)
                    ]

        @pl.loop(0, supers)
        def _(s):
            base = base_w + s * n_super
            pltpu.sync_copy(i_hbm.at[pl.ds(base, n_super)], idx_v)

            @pl.loop(0, n_super, step=_LANES)
            def _(j):
                pidx_v[pl.ds(j, _LANES)] = idx_v[pl.ds(j, _LANES)] >> 2

            for p in range(_NBUF - 1):
                start_gather(p, p)

            @pl.loop(0, _SUPER, step=_NBUF)
            def _(c):
                for u in range(_NBUF):
                    cc = c + u
                    slot = u
                    oslot = u & 1

                    @pl.when(cc + _NBUF - 1 < _SUPER)
                    def _():
                        start_gather(cc + _NBUF - 1, (slot + _NBUF - 1) % _NBUF)

                    @pl.when((s > 0) | (cc >= 2))
                    def _():
                        pltpu.make_async_copy(
                            outs[oslot], o_hbm.at[pl.ds(0, _W * dim)], osems[oslot]
                        ).wait()

                    wait_gather(cc, slot)
                    extract(cc, slot, oslot)
                    pltpu.async_copy(
                        outs[oslot],
                        o_hbm.at[pl.ds((base + cc * _W) * dim, _W * dim)],
                        osems[oslot],
                    )

        for oslot in range(2):
            pltpu.make_async_copy(
                outs[oslot], o_hbm.at[pl.ds(0, _W * dim)], osems[oslot]
            ).wait()

    out = gather_kernel(w_packed, flat_ids)
    return out.reshape(seq, batch, dim).transpose(1, 0, 2)


# speedup vs baseline: 4.8008x; 1.1606x over previous
import jax
import jax.numpy as jnp
from jax import lax
from jax.experimental import pallas as pl
from jax.experimental.pallas import tpu as pltpu
from jax.experimental.pallas import tpu_sc as plsc

_W = 128
_SUPER = 16
_NW = 32
_LANES = 16
_NBUF = 4
_K = 5


def kernel(ids, weight):
    batch, seq = ids.shape
    n_all = batch * seq
    vocab, dim = weight.shape
    pack = 128 // dim
    all_ids = ids.T.reshape(n_all)
    w_packed = weight.reshape(vocab // pack, 128)

    n_ids = n_all // _K
    n_per_w = n_ids // _NW
    n_super = _SUPER * _W
    supers = n_per_w // n_super

    mesh = plsc.VectorSubcoreMesh(core_axis_name="core", subcore_axis_name="subcore")

    @pl.kernel(
        out_type=jax.ShapeDtypeStruct((n_ids * dim,), weight.dtype),
        mesh=mesh,
        scratch_types=[
            pltpu.VMEM((n_super,), jnp.int32),
            pltpu.VMEM((n_super,), jnp.int32),
        ]
        + [pltpu.VMEM((_W, 128), jnp.float32)] * _NBUF
        + [pltpu.VMEM((_W * dim,), jnp.float32)] * 2
        + [pltpu.SemaphoreType.DMA] * (_NBUF + 2),
    )
    def gather_kernel(w_hbm, i_hbm, o_hbm, idx_v, pidx_v, *bufs):
        rows = bufs[:_NBUF]
        outs = bufs[_NBUF : _NBUF + 2]
        gsems = bufs[_NBUF + 2 : 2 * _NBUF + 2]
        osems = bufs[2 * _NBUF + 2 :]
        wid = lax.axis_index("subcore") * 2 + lax.axis_index("core")
        base_w = wid * n_per_w

        def start_gather(c, slot):
            pltpu.async_copy(
                w_hbm.at[pidx_v.at[pl.ds(c * _W, _W)]], rows[slot], gsems[slot]
            )

        def wait_gather(c, slot):
            pltpu.make_async_copy(
                w_hbm.at[pidx_v.at[pl.ds(c * _W, _W)]], rows[slot], gsems[slot]
            ).wait()

        def extract(c, slot, oslot):
            @pl.loop(0, _W, step=_LANES)
            def _(b):
                offs = (idx_v[pl.ds(c * _W + b, _LANES)] & (pack - 1)) * dim
                for j in range(_LANES):
                    r = b + j
                    off = offs[j]
                    outs[oslot][pl.ds(r * dim, _LANES)] = rows[slot][
                        r, pl.ds(off, _LANES)
                    ]
                    outs[oslot][pl.ds(r * dim + _LANES, _LANES)] = rows[slot][
                        r, pl.ds(off + _LANES, _LANES)
                    ]

        @pl.loop(0, supers)
        def _(s):
            base = base_w + s * n_super
            pltpu.sync_copy(i_hbm.at[pl.ds(base, n_super)], idx_v)

            @pl.loop(0, n_super, step=_LANES)
            def _(j):
                pidx_v[pl.ds(j, _LANES)] = idx_v[pl.ds(j, _LANES)] >> 2

            for p in range(_NBUF - 1):
                start_gather(p, p)

            @pl.loop(0, _SUPER, step=_NBUF)
            def _(c):
                for u in range(_NBUF):
                    cc = c + u
                    slot = u
                    oslot = u & 1

                    @pl.when(cc + _NBUF - 1 < _SUPER)
                    def _():
                        start_gather(cc + _NBUF - 1, (slot + _NBUF - 1) % _NBUF)

                    @pl.when((s > 0) | (cc >= 2))
                    def _():
                        pltpu.make_async_copy(
                            outs[oslot], o_hbm.at[pl.ds(0, _W * dim)], osems[oslot]
                        ).wait()

                    wait_gather(cc, slot)
                    extract(cc, slot, oslot)
                    pltpu.async_copy(
                        outs[oslot],
                        o_hbm.at[pl.ds((base + cc * _W) * dim, _W * dim)],
                        osems[oslot],
                    )

        for oslot in range(2):
            pltpu.make_async_copy(
                outs[oslot], o_hbm.at[pl.ds(0, _W * dim)], osems[oslot]
            ).wait()

    seq_per = seq // _K
    slabs = []
    for j in range(_K):
        part = gather_kernel(w_packed, all_ids[j * n_ids : (j + 1) * n_ids])
        slabs.append(part.reshape(seq_per, batch, dim).transpose(1, 0, 2))
    return jnp.concatenate(slabs, axis=1)


# speedup vs baseline: 4.8041x; 1.0007x over previous
import jax
import jax.numpy as jnp
from jax import lax
from jax.experimental import pallas as pl
from jax.experimental.pallas import tpu as pltpu
from jax.experimental.pallas import tpu_sc as plsc

_W = 128
_SUPER = 16
_NW = 32
_LANES = 16
_NBUF = 4
_K = 5


def kernel(ids, weight):
    batch, seq = ids.shape
    n_all = batch * seq
    vocab, dim = weight.shape
    pack = 128 // dim
    all_ids = ids.T.reshape(n_all)
    w_packed = weight.reshape(vocab // pack, 128)

    n_ids = n_all // _K
    n_per_w = n_ids // _NW
    n_super = _SUPER * _W
    supers = n_per_w // n_super

    mesh = plsc.VectorSubcoreMesh(core_axis_name="core", subcore_axis_name="subcore")

    @pl.kernel(
        out_type=jax.ShapeDtypeStruct((n_ids * dim,), weight.dtype),
        mesh=mesh,
        scratch_types=[
            pltpu.VMEM((n_super,), jnp.int32),
            pltpu.VMEM((n_super,), jnp.int32),
        ]
        + [pltpu.VMEM((_W, 128), jnp.float32)] * _NBUF
        + [pltpu.VMEM((_W * dim,), jnp.float32)] * 2
        + [pltpu.SemaphoreType.DMA] * (_NBUF + 2),
    )
    def gather_kernel(w_hbm, i_hbm, o_hbm, idx_v, pidx_v, *bufs):
        rows = bufs[:_NBUF]
        outs = bufs[_NBUF : _NBUF + 2]
        gsems = bufs[_NBUF + 2 : 2 * _NBUF + 2]
        osems = bufs[2 * _NBUF + 2 :]
        wid = lax.axis_index("subcore") * 2 + lax.axis_index("core")
        base_w = wid * n_per_w

        def start_gather(c, slot):
            pltpu.async_copy(
                w_hbm.at[pidx_v.at[pl.ds(c * _W, _W)]], rows[slot], gsems[slot]
            )

        def wait_gather(c, slot):
            pltpu.make_async_copy(
                w_hbm.at[pidx_v.at[pl.ds(c * _W, _W)]], rows[slot], gsems[slot]
            ).wait()

        def extract(c, slot, oslot):
            @pl.loop(0, _W, step=_LANES)
            def _(b):
                offs = (idx_v[pl.ds(c * _W + b, _LANES)] & (pack - 1)) * dim
                for j in range(_LANES):
                    r = b + j
                    off = offs[j]
                    outs[oslot][pl.ds(r * dim, _LANES)] = rows[slot][
                        r, pl.ds(off, _LANES)
                    ]
                    outs[oslot][pl.ds(r * dim + _LANES, _LANES)] = rows[slot][
                        r, pl.ds(off + _LANES, _LANES)
                    ]

        @pl.loop(0, supers)
        def _(s):
            base = base_w + s * n_super
            pltpu.sync_copy(i_hbm.at[pl.ds(base, n_super)], idx_v)

            @pl.loop(0, n_super, step=_LANES)
            def _(j):
                pidx_v[pl.ds(j, _LANES)] = idx_v[pl.ds(j, _LANES)] >> 2

            for p in range(_NBUF - 1):
                start_gather(p, p)

            @pl.loop(0, _SUPER, step=_NBUF)
            def _(c):
                for u in range(_NBUF):
                    cc = c + u
                    slot = u
                    oslot = u & 1

                    @pl.when(cc + _NBUF - 1 < _SUPER)
                    def _():
                        start_gather(cc + _NBUF - 1, (slot + _NBUF - 1) % _NBUF)

                    @pl.when((s > 0) | (cc >= 2))
                    def _():
                        pltpu.make_async_copy(
                            outs[oslot], o_hbm.at[pl.ds(0, _W * dim)], osems[oslot]
                        ).wait()

                    wait_gather(cc, slot)
                    extract(cc, slot, oslot)
                    pltpu.async_copy(
                        outs[oslot],
                        o_hbm.at[pl.ds((base + cc * _W) * dim, _W * dim)],
                        osems[oslot],
                    )

        for oslot in range(2):
            pltpu.make_async_copy(
                outs[oslot], o_hbm.at[pl.ds(0, _W * dim)], osems[oslot]
            ).wait()

    seq_per = seq // _K
    slabs = []
    for j in range(_K):
        part = gather_kernel(w_packed, all_ids[j * n_ids : (j + 1) * n_ids])
        slabs.append(part.reshape(seq_per, batch, dim).transpose(0, 2, 1))
    return jnp.concatenate(slabs, axis=0).transpose(2, 0, 1)
